# baseline (speedup 1.0000x reference)
import functools

import jax
import jax.numpy as jnp
from jax import lax
from jax.experimental import pallas as pl
from jax.experimental.pallas import tpu as pltpu
from jax.experimental.pallas import tpu_sc as plsc

_NC = 2
_NS = 16
_NW = _NC * _NS
_L = 16

_BATCH = 4
_SEQ = 2048
_HIDDEN = 64
_B = _BATCH * _SEQ
_NSC = 4096
_NTC = _B - _NSC
_BPW = _NSC // _NW
_NGRP = _BPW // _L
_HH = _HIDDEN // 2
_CHK = _NTC // 128


def _splat(v, lane):
    dnums = lax.GatherDimensionNumbers(
        offset_dims=(), collapsed_slice_dims=(0,), start_index_map=(0,))
    idx = jnp.full((_L, 1), lane, dtype=jnp.int32)
    return lax.gather(v, idx, dnums, slice_sizes=(1,),
                      mode=lax.GatherScatterMode.PROMISE_IN_BOUNDS)


def _rsqrt(v):
    vi = lax.bitcast_convert_type(v, jnp.int32)
    yi = jnp.int32(0x5F3759DF) - lax.shift_right_logical(vi, 1)
    y = lax.bitcast_convert_type(yi, jnp.float32)
    for _ in range(2):
        y = y * (1.5 - 0.5 * v * y * y)
    return y


def _sc_body(ids_hbm, ttab_hbm, pos_hbm, gamma_hbm, beta_hbm, out_hbm,
             idx_v, x_v, pos_v, gamma_v, beta_v, bank, sem):
    wid = lax.axis_index("s") * _NC + lax.axis_index("c")
    base = wid * _BPW
    pos_base = (wid % (_SEQ // _BPW)) * _BPW

    pltpu.sync_copy(ids_hbm.at[wid], idx_v)
    pltpu.sync_copy(pos_hbm.at[:, pl.ds(pos_base, _BPW)], pos_v)
    pltpu.sync_copy(gamma_hbm, gamma_v)
    pltpu.sync_copy(beta_hbm, beta_v)

    lane_iota = lax.iota(jnp.int32, _L)

    def group(g, carry):
        lanes = pl.ds(g * _L, _L)
        vec = idx_v[0, lanes]
        mvec = lax.bitwise_and(vec, jnp.int32(127))
        for half in range(2):
            h0 = half * _HH
            copies = []
            for l in range(_L):
                q = lax.shift_right_logical(vec[l], 7)
                col = pl.multiple_of(q * 128, 128)
                copies.append(pltpu.make_async_copy(
                    ttab_hbm.at[pl.ds(h0, _HH), pl.ds(col, 128)],
                    bank.at[l],
                    sem,
                ))
            for c in copies:
                c.start()
            for c in copies:
                c.wait()
            for h in range(_HH):
                vals = plsc.load_gather(
                    bank, [lane_iota, jnp.full((_L,), h, jnp.int32), mvec])
                x_v[h0 + h, lanes] = vals
        return carry

    lax.fori_loop(0, _NGRP, group, 0)

    inv_h = jnp.float32(1.0 / _HIDDEN)
    gvec = [gamma_v[pl.ds(k * _L, _L)] for k in range(_HIDDEN // _L)]
    bvec = [beta_v[pl.ds(k * _L, _L)] for k in range(_HIDDEN // _L)]

    def tile(t, carry):
        lanes = pl.ds(t * _L, _L)
        s = x_v[0, lanes] + pos_v[0, lanes]
        ss = s * s
        for h in range(1, _HIDDEN):
            x = x_v[h, lanes] + pos_v[h, lanes]
            s = s + x
            ss = ss + x * x
        mean = s * inv_h
        var = ss * inv_h - mean * mean
        rstd = _rsqrt(var + 1e-12)
        for h in range(_HIDDEN):
            g = _splat(gvec[h // _L], h % _L)
            b = _splat(bvec[h // _L], h % _L)
            x = x_v[h, lanes] + pos_v[h, lanes]
            x_v[h, lanes] = (x - mean) * rstd * g + b
        return carry

    lax.fori_loop(0, _NGRP, tile, 0)

    pltpu.sync_copy(x_v, out_hbm.at[:, pl.ds(base, _BPW)])


def _tc_body(q_ref, m_ref, tab_ref, pos_ref, g_ref, b_ref, out_ref):
    c = pl.program_id(0)
    t = pl.program_id(1)
    m = m_ref[c * 128 + t]
    shift = jnp.remainder(t - m, 128)
    rolled = pltpu.roll(tab_ref[...], shift, 1)
    lane = lax.broadcasted_iota(jnp.int32, (_HIDDEN, 128), 1)
    acc = jnp.where(lane == t, rolled, out_ref[...])

    @pl.when(t < 127)
    def _store():
        out_ref[...] = acc

    @pl.when(t == 127)
    def _finish():
        x = acc + pos_ref[...]
        mean = jnp.mean(x, axis=0, keepdims=True)
        d = x - mean
        var = jnp.mean(d * d, axis=0, keepdims=True)
        out_ref[...] = d * lax.rsqrt(var + 1e-12) * g_ref[...] + b_ref[...]


@jax.jit
def _embed_ln(ids_sc, q_tc, m_tc, ttab, post, gamma, beta):
    mesh = plsc.VectorSubcoreMesh(core_axis_name="c", subcore_axis_name="s")
    sc_kern = functools.partial(
        pl.kernel,
        out_type=jax.ShapeDtypeStruct((_HIDDEN, _NSC), jnp.float32),
        mesh=mesh,
        scratch_types=[
            pltpu.VMEM((1, _BPW), jnp.int32),
            pltpu.VMEM((_HIDDEN, _BPW), jnp.float32),
            pltpu.VMEM((_HIDDEN, _BPW), jnp.float32),
            pltpu.VMEM((_HIDDEN,), jnp.float32),
            pltpu.VMEM((_HIDDEN,), jnp.float32),
            pltpu.VMEM((_L, _HH, 128), jnp.float32),
            pltpu.SemaphoreType.DMA,
        ],
        compiler_params=pltpu.CompilerParams(
            use_tc_tiling_on_sc=True, needs_layout_passes=False),
    )(_sc_body)
    sc_out = sc_kern(ids_sc, ttab, post, gamma, beta)

    g2 = gamma.reshape(_HIDDEN, 1)
    b2 = beta.reshape(_HIDDEN, 1)
    s128 = _NSC // 128
    nposb = _SEQ // 128
    tc_out = pl.pallas_call(
        _tc_body,
        grid_spec=pltpu.PrefetchScalarGridSpec(
            num_scalar_prefetch=2,
            grid=(_CHK, 128),
            in_specs=[
                pl.BlockSpec((_HIDDEN, 128),
                             lambda c, t, q, m: (0, q[c * 128 + t])),
                pl.BlockSpec((_HIDDEN, 128),
                             lambda c, t, q, m: (0, (s128 + c) % nposb)),
                pl.BlockSpec((_HIDDEN, 1), lambda c, t, q, m: (0, 0)),
                pl.BlockSpec((_HIDDEN, 1), lambda c, t, q, m: (0, 0)),
            ],
            out_specs=pl.BlockSpec((_HIDDEN, 128),
                                   lambda c, t, q, m: (0, c)),
        ),
        out_shape=jax.ShapeDtypeStruct((_HIDDEN, _NTC), jnp.float32),
    )(q_tc, m_tc, ttab, post, g2, b2)

    return jnp.concatenate([sc_out, tc_out], axis=1)


def kernel(input_ids, token_table, pos_table, gamma, beta):
    ids = input_ids.astype(jnp.int32).reshape(_B)
    ids_sc = ids[:_NSC].reshape(_NW, 1, _BPW)
    q_tc = lax.shift_right_logical(ids[_NSC:], 7)
    m_tc = lax.bitwise_and(ids[_NSC:], 127)
    ttab = token_table.T
    post = pos_table.T
    out_t = _embed_ln(ids_sc, q_tc, m_tc, ttab, post, gamma, beta)
    return out_t.T.reshape(_BATCH, _SEQ, _HIDDEN)


# speedup vs baseline: 16.4018x; 16.4018x over previous
import functools

import jax
import jax.numpy as jnp
from jax import lax
from jax.experimental import pallas as pl
from jax.experimental.pallas import tpu as pltpu
from jax.experimental.pallas import tpu_sc as plsc

_NC = 2
_NS = 16
_NW = _NC * _NS
_L = 16

_BATCH = 4
_SEQ = 2048
_HIDDEN = 64
_B = _BATCH * _SEQ
_NSC = 4096
_NTC = _B - _NSC
_BPW = _NSC // _NW
_NGRP = _BPW // _L
_HH = _HIDDEN // 2
_CHK = _NTC // 128


def _splat(v, lane):
    dnums = lax.GatherDimensionNumbers(
        offset_dims=(), collapsed_slice_dims=(0,), start_index_map=(0,))
    idx = jnp.full((_L, 1), lane, dtype=jnp.int32)
    return lax.gather(v, idx, dnums, slice_sizes=(1,),
                      mode=lax.GatherScatterMode.PROMISE_IN_BOUNDS)


def _rsqrt(v):
    vi = lax.bitcast_convert_type(v, jnp.int32)
    yi = jnp.int32(0x5F3759DF) - lax.shift_right_logical(vi, 1)
    y = lax.bitcast_convert_type(yi, jnp.float32)
    for _ in range(2):
        y = y * (1.5 - 0.5 * v * y * y)
    return y


def _sc_body(ids_hbm, ttab_hbm, pos_hbm, gamma_hbm, beta_hbm, out_hbm,
             idx_v, x_v, pos_v, gamma_v, beta_v, bank, sem):
    wid = lax.axis_index("s") * _NC + lax.axis_index("c")
    base = wid * _BPW
    pos_base = (wid % (_SEQ // _BPW)) * _BPW

    pltpu.sync_copy(ids_hbm.at[wid], idx_v)
    pltpu.sync_copy(pos_hbm.at[:, pl.ds(pos_base, _BPW)], pos_v)
    pltpu.sync_copy(gamma_hbm, gamma_v)
    pltpu.sync_copy(beta_hbm, beta_v)

    lane_iota = lax.iota(jnp.int32, _L)

    def group(g, carry):
        lanes = pl.ds(g * _L, _L)
        vec = idx_v[0, lanes]
        mvec = lax.bitwise_and(vec, jnp.int32(127))
        for half in range(2):
            h0 = half * _HH
            copies = []
            for l in range(_L):
                q = lax.shift_right_logical(vec[l], 7)
                col = pl.multiple_of(q * 128, 128)
                copies.append(pltpu.make_async_copy(
                    ttab_hbm.at[pl.ds(h0, _HH), pl.ds(col, 128)],
                    bank.at[l],
                    sem,
                ))
            for c in copies:
                c.start()
            for c in copies:
                c.wait()
            for h in range(_HH):
                vals = plsc.load_gather(
                    bank, [lane_iota, jnp.full((_L,), h, jnp.int32), mvec])
                x_v[h0 + h, lanes] = vals
        return carry

    lax.fori_loop(0, _NGRP, group, 0)

    inv_h = jnp.float32(1.0 / _HIDDEN)
    gvec = [gamma_v[pl.ds(k * _L, _L)] for k in range(_HIDDEN // _L)]
    bvec = [beta_v[pl.ds(k * _L, _L)] for k in range(_HIDDEN // _L)]

    def tile(t, carry):
        lanes = pl.ds(t * _L, _L)
        s = x_v[0, lanes] + pos_v[0, lanes]
        ss = s * s
        for h in range(1, _HIDDEN):
            x = x_v[h, lanes] + pos_v[h, lanes]
            s = s + x
            ss = ss + x * x
        mean = s * inv_h
        var = ss * inv_h - mean * mean
        rstd = _rsqrt(var + 1e-12)
        for h in range(_HIDDEN):
            g = _splat(gvec[h // _L], h % _L)
            b = _splat(bvec[h // _L], h % _L)
            x = x_v[h, lanes] + pos_v[h, lanes]
            x_v[h, lanes] = (x - mean) * rstd * g + b
        return carry

    lax.fori_loop(0, _NGRP, tile, 0)

    pltpu.sync_copy(x_v, out_hbm.at[:, pl.ds(base, _BPW)])


def _tc_fire(q_ref, tab_ref, banks, sems, chunk, buf):
    for l in range(128):
        q = q_ref[chunk * 128 + l]
        pltpu.make_async_copy(
            tab_ref.at[:, pl.ds(q * 128, 128)],
            banks.at[buf, l],
            sems.at[buf],
        ).start()


def _tc_drain(q_ref, tab_ref, banks, sems, chunk, buf):
    for l in range(128):
        q = q_ref[chunk * 128 + l]
        pltpu.make_async_copy(
            tab_ref.at[:, pl.ds(q * 128, 128)],
            banks.at[buf, l],
            sems.at[buf],
        ).wait()


def _tc_body(q_ref, m_ref, tab_ref, pos_ref, g_ref, b_ref, out_ref,
             banks, sems):
    c = pl.program_id(0)
    buf = lax.rem(c, 2)

    @pl.when(c == 0)
    def _prime():
        _tc_fire(q_ref, tab_ref, banks, sems, c, buf)

    @pl.when(c + 1 < _CHK)
    def _ahead():
        _tc_fire(q_ref, tab_ref, banks, sems, c + 1, lax.rem(c + 1, 2))

    _tc_drain(q_ref, tab_ref, banks, sems, c, buf)

    lane = lax.broadcasted_iota(jnp.int32, (_HIDDEN, 128), 1)
    acc = jnp.zeros((_HIDDEN, 128), jnp.float32)
    for l in range(128):
        m = m_ref[c * 128 + l]
        shift = jnp.remainder(l - m, 128)
        rolled = pltpu.roll(banks[buf, l], shift, 1)
        acc = jnp.where(lane == l, rolled, acc)

    x = acc + pos_ref[...]
    mean = jnp.mean(x, axis=0, keepdims=True)
    d = x - mean
    var = jnp.mean(d * d, axis=0, keepdims=True)
    out_ref[...] = d * lax.rsqrt(var + 1e-12) * g_ref[...] + b_ref[...]


@jax.jit
def _embed_ln(ids_sc, q_tc, m_tc, ttab, post, gamma, beta):
    mesh = plsc.VectorSubcoreMesh(core_axis_name="c", subcore_axis_name="s")
    sc_kern = functools.partial(
        pl.kernel,
        out_type=jax.ShapeDtypeStruct((_HIDDEN, _NSC), jnp.float32),
        mesh=mesh,
        scratch_types=[
            pltpu.VMEM((1, _BPW), jnp.int32),
            pltpu.VMEM((_HIDDEN, _BPW), jnp.float32),
            pltpu.VMEM((_HIDDEN, _BPW), jnp.float32),
            pltpu.VMEM((_HIDDEN,), jnp.float32),
            pltpu.VMEM((_HIDDEN,), jnp.float32),
            pltpu.VMEM((_L, _HH, 128), jnp.float32),
            pltpu.SemaphoreType.DMA,
        ],
        compiler_params=pltpu.CompilerParams(
            use_tc_tiling_on_sc=True, needs_layout_passes=False),
    )(_sc_body)
    sc_out = sc_kern(ids_sc, ttab, post, gamma, beta)

    g2 = gamma.reshape(_HIDDEN, 1)
    b2 = beta.reshape(_HIDDEN, 1)
    s128 = _NSC // 128
    nposb = _SEQ // 128
    tc_out = pl.pallas_call(
        _tc_body,
        grid_spec=pltpu.PrefetchScalarGridSpec(
            num_scalar_prefetch=2,
            grid=(_CHK,),
            in_specs=[
                pl.BlockSpec(memory_space=pl.ANY),
                pl.BlockSpec((_HIDDEN, 128),
                             lambda c, q, m: (0, (s128 + c) % nposb)),
                pl.BlockSpec((_HIDDEN, 1), lambda c, q, m: (0, 0)),
                pl.BlockSpec((_HIDDEN, 1), lambda c, q, m: (0, 0)),
            ],
            out_specs=pl.BlockSpec((_HIDDEN, 128), lambda c, q, m: (0, c)),
            scratch_shapes=[
                pltpu.VMEM((2, 128, _HIDDEN, 128), jnp.float32),
                pltpu.SemaphoreType.DMA((2,)),
            ],
        ),
        out_shape=jax.ShapeDtypeStruct((_HIDDEN, _NTC), jnp.float32),
    )(q_tc, m_tc, ttab, post, g2, b2)

    return jnp.concatenate([sc_out, tc_out], axis=1)


def kernel(input_ids, token_table, pos_table, gamma, beta):
    ids = input_ids.astype(jnp.int32).reshape(_B)
    ids_sc = ids[:_NSC].reshape(_NW, 1, _BPW)
    q_tc = lax.shift_right_logical(ids[_NSC:], 7)
    m_tc = lax.bitwise_and(ids[_NSC:], 127)
    ttab = token_table.T
    post = pos_table.T
    out_t = _embed_ln(ids_sc, q_tc, m_tc, ttab, post, gamma, beta)
    return out_t.T.reshape(_BATCH, _SEQ, _HIDDEN)
